# baseline (speedup 1.0000x reference)
import numpy as np

import jax
import jax.numpy as jnp
from jax.experimental import pallas as pl
from jax.experimental.pallas import tpu as pltpu
from jax.sharding import Mesh, PartitionSpec as P

NUM_LAYERS = 4
D = 2048
BATCH = 4096
TILE_B = 256


def _make_layer_kernel(has_cbuf):
    def _layer_kernel(*refs):
        if has_cbuf:
            x_ref, c0_ref, w_ref, b_ref, _cbuf_ref, h_ref, c1_ref = refs
        else:
            x_ref, c0_ref, w_ref, b_ref, h_ref, c1_ref = refs
        xb = x_ref[...]
        u = jnp.dot(xb, w_ref[...], preferred_element_type=jnp.float32)
        f = 0.5 * jnp.tanh(0.5 * (u[:, D:2 * D] + b_ref[0, :])) + 0.5
        r = 0.5 * jnp.tanh(0.5 * (u[:, 2 * D:] + b_ref[1, :])) + 0.5
        c1 = f * c0_ref[...] + (1.0 - f) * u[:, :D]
        h = r * jnp.tanh(c1) + (1.0 - r) * xb.astype(jnp.float32)
        h_ref[...] = h.astype(h_ref.dtype)
        c1_ref[0] = c1
    return _layer_kernel


def _layer(layer_idx, x_bf16, c0, w_bf16, b2, cbuf, h_dtype):
    batch = x_bf16.shape[0]
    nb = batch // TILE_B
    in_specs = [
        pl.BlockSpec((TILE_B, D), lambda i: (i, 0)),
        pl.BlockSpec((TILE_B, D), lambda i: (i, 0)),
        pl.BlockSpec((D, 3 * D), lambda i: (0, 0)),
        pl.BlockSpec((2, D), lambda i: (0, 0)),
    ]
    args = [x_bf16, c0, w_bf16, b2]
    aliases = {}
    if cbuf is not None:
        in_specs.append(pl.BlockSpec((1, 8, 128), lambda i: (0, 0, 0)))
        args.append(cbuf)
        aliases = {4: 1}
    return pl.pallas_call(
        _make_layer_kernel(cbuf is not None),
        grid=(nb,),
        in_specs=in_specs,
        out_specs=[
            pl.BlockSpec((TILE_B, D), lambda i: (i, 0)),
            pl.BlockSpec((1, TILE_B, D), lambda i: (layer_idx, i, 0)),
        ],
        out_shape=[
            jax.ShapeDtypeStruct((batch, D), h_dtype),
            jax.ShapeDtypeStruct((NUM_LAYERS, batch, D), jnp.float32),
        ],
        input_output_aliases=aliases,
        compiler_params=pltpu.CompilerParams(
            dimension_semantics=("arbitrary",),
        ),
    )(*args)


def _stack(x_bf16, c0s, w16s, b2s):
    h = x_bf16
    cbuf = None
    for i in range(NUM_LAYERS):
        h_dtype = jnp.float32 if i == NUM_LAYERS - 1 else jnp.bfloat16
        h, cbuf = _layer(i, h, c0s[i], w16s[i], b2s[i], cbuf, h_dtype)
    return h, cbuf


def kernel(input, c_0, W0, b0, W1, b1, W2, b2, W3, b3):
    Ws = [W0, W1, W2, W3]
    bs = [b0, b1, b2, b3]
    x_bf16 = input.astype(jnp.bfloat16)
    c0s = tuple(c_0[i] for i in range(NUM_LAYERS))
    w16s = tuple(w.astype(jnp.bfloat16) for w in Ws)
    b2s = tuple(b.reshape(2, D) for b in bs)

    devs = jax.devices()
    ndev = len(devs)
    while ndev > 1 and (BATCH // TILE_B) % ndev != 0:
        ndev -= 1
    if ndev == 1:
        return _stack(x_bf16, c0s, w16s, b2s)
    mesh = Mesh(np.array(devs[:ndev]), ("b",))
    fn = jax.shard_map(
        _stack,
        mesh=mesh,
        in_specs=(P("b", None), (P("b", None),) * NUM_LAYERS,
                  (P(),) * NUM_LAYERS, (P(),) * NUM_LAYERS),
        out_specs=(P("b", None), P(None, "b", None)),
        check_vma=False,
    )
    return fn(x_bf16, c0s, w16s, b2s)


# speedup vs baseline: 1.6011x; 1.6011x over previous
import jax
import jax.numpy as jnp
from jax.experimental import pallas as pl
from jax.experimental.pallas import tpu as pltpu

NUM_LAYERS = 4
D = 2048
BATCH = 4096
TILE_B = 256
NB = BATCH // TILE_B
WROWS = D // NB


def _make_layer_kernel(has_cbuf, has_wnext):
    def _layer_kernel(*refs):
        refs = list(refs)
        x_ref = refs.pop(0)
        c0_ref = refs.pop(0)
        w_ref = refs.pop(0)
        b_ref = refs.pop(0)
        wn_ref = refs.pop(0) if has_wnext else None
        if has_cbuf:
            refs.pop(0)
        h_ref = refs.pop(0)
        c1_ref = refs.pop(0)
        wn16_ref = refs.pop(0) if has_wnext else None

        xb = x_ref[...]
        u = jnp.dot(xb, w_ref[...], preferred_element_type=jnp.float32)
        f = 0.5 * jnp.tanh(0.5 * (u[:, D:2 * D] + b_ref[0, :])) + 0.5
        r = 0.5 * jnp.tanh(0.5 * (u[:, 2 * D:] + b_ref[1, :])) + 0.5
        c1 = f * c0_ref[...] + (1.0 - f) * u[:, :D]
        h = r * jnp.tanh(c1) + (1.0 - r) * xb.astype(jnp.float32)
        h_ref[...] = h.astype(h_ref.dtype)
        c1_ref[0] = c1
        if has_wnext:
            wn16_ref[...] = wn_ref[...].astype(jnp.bfloat16)
    return _layer_kernel


def _layer(layer_idx, x_bf16, c0, w_bf16, b2, cbuf, w_next_f32, h_dtype):
    has_wnext = w_next_f32 is not None
    in_specs = [
        pl.BlockSpec((TILE_B, D), lambda i: (i, 0)),
        pl.BlockSpec((TILE_B, D), lambda i: (i, 0)),
        pl.BlockSpec((D, 3 * D), lambda i: (0, 0)),
        pl.BlockSpec((2, D), lambda i: (0, 0)),
    ]
    args = [x_bf16, c0, w_bf16, b2]
    if has_wnext:
        in_specs.append(pl.BlockSpec((WROWS, 3 * D), lambda i: (i, 0)))
        args.append(w_next_f32)
    aliases = {}
    if cbuf is not None:
        aliases = {len(args): 1}
        in_specs.append(pl.BlockSpec((1, 8, 128), lambda i: (0, 0, 0)))
        args.append(cbuf)
    out_specs = [
        pl.BlockSpec((TILE_B, D), lambda i: (i, 0)),
        pl.BlockSpec((1, TILE_B, D), lambda i: (layer_idx, i, 0)),
    ]
    out_shape = [
        jax.ShapeDtypeStruct((BATCH, D), h_dtype),
        jax.ShapeDtypeStruct((NUM_LAYERS, BATCH, D), jnp.float32),
    ]
    if has_wnext:
        out_specs.append(pl.BlockSpec((WROWS, 3 * D), lambda i: (i, 0)))
        out_shape.append(jax.ShapeDtypeStruct((D, 3 * D), jnp.bfloat16))
    return pl.pallas_call(
        _make_layer_kernel(cbuf is not None, has_wnext),
        grid=(NB,),
        in_specs=in_specs,
        out_specs=out_specs,
        out_shape=out_shape,
        input_output_aliases=aliases,
        compiler_params=pltpu.CompilerParams(
            dimension_semantics=("arbitrary",),
        ),
    )(*args)


def kernel(input, c_0, W0, b0, W1, b1, W2, b2, W3, b3):
    Ws = [W0, W1, W2, W3]
    bs = [b0, b1, b2, b3]
    h = input.astype(jnp.bfloat16)
    w16 = Ws[0].astype(jnp.bfloat16)
    cbuf = None
    for i in range(NUM_LAYERS):
        last = i == NUM_LAYERS - 1
        h_dtype = jnp.float32 if last else jnp.bfloat16
        w_next = None if last else Ws[i + 1]
        out = _layer(i, h, c_0[i], w16, bs[i].reshape(2, D), cbuf,
                     w_next, h_dtype)
        if last:
            h, cbuf = out
        else:
            h, cbuf, w16 = out
    return (h, cbuf)


# speedup vs baseline: 1.6246x; 1.0147x over previous
import jax
import jax.numpy as jnp
from jax.experimental import pallas as pl
from jax.experimental.pallas import tpu as pltpu

NUM_LAYERS = 4
D = 2048
BATCH = 4096
TILE_B = 256
NB = BATCH // TILE_B
WROWS = D // NB


def _make_layer_kernel(has_cbuf, has_wnext):
    def _layer_kernel(*refs):
        refs = list(refs)
        x_ref = refs.pop(0)
        c0_ref = refs.pop(0)
        w_ref = refs.pop(0)
        b_ref = refs.pop(0)
        wn_ref = refs.pop(0) if has_wnext else None
        if has_cbuf:
            refs.pop(0)
        h_ref = refs.pop(0)
        c1_ref = refs.pop(0)
        wn16_ref = refs.pop(0) if has_wnext else None

        xb = x_ref[...]
        u = jnp.dot(xb, w_ref[...], preferred_element_type=jnp.float32)
        f = 0.5 * jnp.tanh(0.5 * (u[:, D:2 * D] + b_ref[0, :])) + 0.5
        r = 0.5 * jnp.tanh(0.5 * (u[:, 2 * D:] + b_ref[1, :])) + 0.5
        xt = u[:, :D]
        c1 = xt + f * (c0_ref[...] - xt)
        x32 = xb.astype(jnp.float32)
        h = x32 + r * (jnp.tanh(c1) - x32)
        h_ref[...] = h.astype(h_ref.dtype)
        c1_ref[0] = c1
        if has_wnext:
            wn16_ref[...] = wn_ref[...].astype(jnp.bfloat16)
    return _layer_kernel


def _layer(layer_idx, x_bf16, c0, w_bf16, b2, cbuf, w_next_f32, h_dtype):
    has_wnext = w_next_f32 is not None
    in_specs = [
        pl.BlockSpec((TILE_B, D), lambda i: (i, 0)),
        pl.BlockSpec((TILE_B, D), lambda i: (i, 0)),
        pl.BlockSpec((D, 3 * D), lambda i: (0, 0)),
        pl.BlockSpec((2, D), lambda i: (0, 0)),
    ]
    args = [x_bf16, c0, w_bf16, b2]
    if has_wnext:
        in_specs.append(pl.BlockSpec((WROWS, 3 * D), lambda i: (i, 0)))
        args.append(w_next_f32)
    aliases = {}
    if cbuf is not None:
        aliases = {len(args): 1}
        in_specs.append(pl.BlockSpec((1, 8, 128), lambda i: (0, 0, 0)))
        args.append(cbuf)
    out_specs = [
        pl.BlockSpec((TILE_B, D), lambda i: (i, 0)),
        pl.BlockSpec((1, TILE_B, D), lambda i: (layer_idx, i, 0)),
    ]
    out_shape = [
        jax.ShapeDtypeStruct((BATCH, D), h_dtype),
        jax.ShapeDtypeStruct((NUM_LAYERS, BATCH, D), jnp.float32),
    ]
    if has_wnext:
        out_specs.append(pl.BlockSpec((WROWS, 3 * D), lambda i: (i, 0)))
        out_shape.append(jax.ShapeDtypeStruct((D, 3 * D), jnp.bfloat16))
    return pl.pallas_call(
        _make_layer_kernel(cbuf is not None, has_wnext),
        grid=(NB,),
        in_specs=in_specs,
        out_specs=out_specs,
        out_shape=out_shape,
        input_output_aliases=aliases,
        compiler_params=pltpu.CompilerParams(
            dimension_semantics=("arbitrary",),
        ),
    )(*args)


def kernel(input, c_0, W0, b0, W1, b1, W2, b2, W3, b3):
    Ws = [W0, W1, W2, W3]
    bs = [b0, b1, b2, b3]
    h = input.astype(jnp.bfloat16)
    w16 = Ws[0].astype(jnp.bfloat16)
    cbuf = None
    for i in range(NUM_LAYERS):
        last = i == NUM_LAYERS - 1
        h_dtype = jnp.float32 if last else jnp.bfloat16
        w_next = None if last else Ws[i + 1]
        out = _layer(i, h, c_0[i], w16, bs[i].reshape(2, D), cbuf,
                     w_next, h_dtype)
        if last:
            h, cbuf = out
        else:
            h, cbuf, w16 = out
    return (h, cbuf)


# speedup vs baseline: 1.6597x; 1.0216x over previous
import jax
import jax.numpy as jnp
from jax.experimental import pallas as pl
from jax.experimental.pallas import tpu as pltpu

NUM_LAYERS = 4
D = 2048
BATCH = 4096
TILE_B = 256
NB = BATCH // TILE_B
WROWS = D // NB


def _make_layer_kernel(has_cbuf, has_wnext, x_is_f32):
    def _layer_kernel(*refs):
        refs = list(refs)
        x_ref = refs.pop(0)
        c0_ref = refs.pop(0)
        w_ref = refs.pop(0)
        b_ref = refs.pop(0)
        wn_ref = refs.pop(0) if has_wnext else None
        if has_cbuf:
            refs.pop(0)
        h_ref = refs.pop(0)
        c1_ref = refs.pop(0)
        wn16_ref = refs.pop(0) if has_wnext else None

        if x_is_f32:
            x32 = x_ref[...]
            xb = x32.astype(jnp.bfloat16)
        else:
            xb = x_ref[...]
            x32 = xb.astype(jnp.float32)
        u = jnp.dot(xb, w_ref[...], preferred_element_type=jnp.float32)
        f = 0.5 * jnp.tanh(0.5 * (u[:, D:2 * D] + b_ref[0, :])) + 0.5
        r = 0.5 * jnp.tanh(0.5 * (u[:, 2 * D:] + b_ref[1, :])) + 0.5
        xt = u[:, :D]
        c1 = xt + f * (c0_ref[...] - xt)
        h = x32 + r * (jnp.tanh(c1) - x32)
        h_ref[...] = h.astype(h_ref.dtype)
        c1_ref[0] = c1
        if has_wnext:
            wn16_ref[...] = wn_ref[...].astype(jnp.bfloat16)
    return _layer_kernel


def _layer(layer_idx, x, c0, w_bf16, b2, cbuf, w_next_f32, h_dtype):
    has_wnext = w_next_f32 is not None
    x_is_f32 = x.dtype == jnp.float32
    in_specs = [
        pl.BlockSpec((TILE_B, D), lambda i: (i, 0)),
        pl.BlockSpec((TILE_B, D), lambda i: (i, 0)),
        pl.BlockSpec((D, 3 * D), lambda i: (0, 0)),
        pl.BlockSpec((2, D), lambda i: (0, 0)),
    ]
    args = [x, c0, w_bf16, b2]
    if has_wnext:
        in_specs.append(pl.BlockSpec((WROWS, 3 * D), lambda i: (i, 0)))
        args.append(w_next_f32)
    aliases = {}
    if cbuf is not None:
        aliases = {len(args): 1}
        in_specs.append(pl.BlockSpec((1, 8, 128), lambda i: (0, 0, 0)))
        args.append(cbuf)
    out_specs = [
        pl.BlockSpec((TILE_B, D), lambda i: (i, 0)),
        pl.BlockSpec((1, TILE_B, D), lambda i: (layer_idx, i, 0)),
    ]
    out_shape = [
        jax.ShapeDtypeStruct((BATCH, D), h_dtype),
        jax.ShapeDtypeStruct((NUM_LAYERS, BATCH, D), jnp.float32),
    ]
    if has_wnext:
        out_specs.append(pl.BlockSpec((WROWS, 3 * D), lambda i: (i, 0)))
        out_shape.append(jax.ShapeDtypeStruct((D, 3 * D), jnp.bfloat16))
    return pl.pallas_call(
        _make_layer_kernel(cbuf is not None, has_wnext, x_is_f32),
        grid=(NB,),
        in_specs=in_specs,
        out_specs=out_specs,
        out_shape=out_shape,
        input_output_aliases=aliases,
        compiler_params=pltpu.CompilerParams(
            dimension_semantics=("arbitrary",),
        ),
    )(*args)


def kernel(input, c_0, W0, b0, W1, b1, W2, b2, W3, b3):
    Ws = [W0, W1, W2, W3]
    bs = [b0, b1, b2, b3]
    h = input
    w16 = Ws[0].astype(jnp.bfloat16)
    cbuf = None
    for i in range(NUM_LAYERS):
        last = i == NUM_LAYERS - 1
        h_dtype = jnp.float32 if last else jnp.bfloat16
        w_next = None if last else Ws[i + 1]
        out = _layer(i, h, c_0[i], w16, bs[i].reshape(2, D), cbuf,
                     w_next, h_dtype)
        if last:
            h, cbuf = out
        else:
            h, cbuf, w16 = out
    return (h, cbuf)


# speedup vs baseline: 1.6603x; 1.0004x over previous
import jax
import jax.numpy as jnp
from jax.experimental import pallas as pl
from jax.experimental.pallas import tpu as pltpu

NUM_LAYERS = 4
D = 2048
BATCH = 4096
TILE_B = 256
NB = BATCH // TILE_B
WROWS = D // NB


def _make_layer_kernel(has_cbuf, has_wnext, x_is_f32):
    def _layer_kernel(*refs):
        refs = list(refs)
        x_ref = refs.pop(0)
        c0_ref = refs.pop(0)
        w_ref = refs.pop(0)
        b_ref = refs.pop(0)
        wn_ref = refs.pop(0) if has_wnext else None
        if has_cbuf:
            refs.pop(0)
        h_ref = refs.pop(0)
        c1_ref = refs.pop(0)
        wn16_ref = refs.pop(0) if has_wnext else None

        if x_is_f32:
            x32 = x_ref[...]
            xb = x32.astype(jnp.bfloat16)
        else:
            xb = x_ref[...]
            x32 = xb.astype(jnp.float32)
        u = jnp.dot(xb, w_ref[...], preferred_element_type=jnp.float32)
        f = 0.5 * jnp.tanh(0.5 * (u[:, D:2 * D] + b_ref[0, :])) + 0.5
        r = 0.5 * jnp.tanh(0.5 * (u[:, 2 * D:] + b_ref[1, :])) + 0.5
        xt = u[:, :D]
        c1 = xt + f * (c0_ref[...] - xt)
        h = x32 + r * (jnp.tanh(c1) - x32)
        h_ref[...] = h.astype(h_ref.dtype)
        c1_ref[0] = c1
        if has_wnext:
            wn16_ref[...] = wn_ref[...].astype(jnp.bfloat16)
    return _layer_kernel


def _layer(layer_idx, x, c0, w_bf16, b2, cbuf, w_next_f32, h_dtype):
    has_wnext = w_next_f32 is not None
    x_is_f32 = x.dtype == jnp.float32
    in_specs = [
        pl.BlockSpec((TILE_B, D), lambda i: (i, 0)),
        pl.BlockSpec((TILE_B, D), lambda i: (i, 0)),
        pl.BlockSpec((D, 3 * D), lambda i: (0, 0)),
        pl.BlockSpec((2, D), lambda i: (0, 0)),
    ]
    args = [x, c0, w_bf16, b2]
    if has_wnext:
        in_specs.append(pl.BlockSpec((WROWS, 3 * D), lambda i: (i, 0)))
        args.append(w_next_f32)
    aliases = {}
    if cbuf is not None:
        aliases = {len(args): 1}
        in_specs.append(pl.BlockSpec((1, 8, 128), lambda i: (0, 0, 0)))
        args.append(cbuf)
    out_specs = [
        pl.BlockSpec((TILE_B, D), lambda i: (i, 0)),
        pl.BlockSpec((1, TILE_B, D), lambda i: (layer_idx, i, 0)),
    ]
    out_shape = [
        jax.ShapeDtypeStruct((BATCH, D), h_dtype),
        jax.ShapeDtypeStruct((NUM_LAYERS, BATCH, D), jnp.float32),
    ]
    if has_wnext:
        out_specs.append(pl.BlockSpec((WROWS, 3 * D), lambda i: (i, 0)))
        out_shape.append(jax.ShapeDtypeStruct((D, 3 * D), jnp.bfloat16))
    return pl.pallas_call(
        _make_layer_kernel(cbuf is not None, has_wnext, x_is_f32),
        grid=(NB,),
        in_specs=in_specs,
        out_specs=out_specs,
        out_shape=out_shape,
        input_output_aliases=aliases,
        compiler_params=pltpu.CompilerParams(
            dimension_semantics=("parallel",),
        ),
    )(*args)


def kernel(input, c_0, W0, b0, W1, b1, W2, b2, W3, b3):
    Ws = [W0, W1, W2, W3]
    bs = [b0, b1, b2, b3]
    h = input
    w16 = Ws[0].astype(jnp.bfloat16)
    cbuf = None
    for i in range(NUM_LAYERS):
        last = i == NUM_LAYERS - 1
        h_dtype = jnp.float32 if last else jnp.bfloat16
        w_next = None if last else Ws[i + 1]
        out = _layer(i, h, c_0[i], w16, bs[i].reshape(2, D), cbuf,
                     w_next, h_dtype)
        if last:
            h, cbuf = out
        else:
            h, cbuf, w16 = out
    return (h, cbuf)
